# baseline (speedup 1.0000x reference)
import functools

import jax
import jax.numpy as jnp
from jax import lax
from jax.experimental import pallas as pl
from jax.experimental.pallas import tpu as pltpu
from jax.experimental.pallas import tpu_sc as plsc

N_AGENTS = 1000
N_TASKS = 9000
N_NODES = 10000
N_EDGES = 320000
H = 128
NUM_LAYERS = 3

NC = 2
NS = 16
EDGES_PER_TILE = N_EDGES // (NC * NS)
CHUNK = 80
NCHUNK = EDGES_PER_TILE // CHUNK
N_PAD = 10240
ROWS_PER_TILE = N_PAD // NS
WB = 128
NWB = ROWS_PER_TILE // WB
NRB = 4
NIB = 8
LG = 2
LI = 6
WBC = ROWS_PER_TILE // CHUNK


def _sc_agg_body(x_hbm, src_hbm, dst_hbm, out_hbm,
                 srcc, dstc, rows, agg_sh, isem, gsem, ssem):
    c = lax.axis_index("c")
    sid = lax.axis_index("s")
    ebase = (c * NS + sid) * EDGES_PER_TILE
    rbase = sid * ROWS_PER_TILE

    z16 = jnp.zeros((16,), jnp.float32)

    def zrow(r, carry):
        for j in range(8):
            rows[0, r, pl.ds(j * 16, 16)] = z16
        return carry
    lax.fori_loop(0, CHUNK, zrow, 0)

    for k in range(WBC):
        pltpu.async_copy(rows.at[0], agg_sh.at[pl.ds(rbase + k * CHUNK, CHUNK)],
                         gsem.at[0])

    def idx_load(t):
        ib = lax.rem(t, NIB)
        pltpu.async_copy(src_hbm.at[pl.ds(ebase + t * CHUNK, CHUNK)],
                         srcc.at[ib], isem.at[ib])
        pltpu.async_copy(dst_hbm.at[pl.ds(ebase + t * CHUNK, CHUNK)],
                         dstc.at[ib], isem.at[ib])

    def idx_wait(t):
        ib = lax.rem(t, NIB)
        pltpu.make_async_copy(src_hbm.at[pl.ds(0, CHUNK)],
                              srcc.at[ib], isem.at[ib]).wait()
        pltpu.make_async_copy(dst_hbm.at[pl.ds(0, CHUNK)],
                              dstc.at[ib], isem.at[ib]).wait()

    def gather_start(t):
        ib = lax.rem(t, NIB)
        rb = lax.rem(t, NRB)
        pltpu.async_copy(x_hbm.at[srcc.at[ib]], rows.at[rb], gsem.at[rb])

    def gather_wait(t):
        rb = lax.rem(t, NRB)
        pltpu.make_async_copy(x_hbm.at[srcc.at[0]], rows.at[rb],
                              gsem.at[rb]).wait()

    def scatter_start(t):
        ib = lax.rem(t, NIB)
        rb = lax.rem(t, NRB)
        pltpu.async_copy(rows.at[rb], agg_sh.at[dstc.at[ib]], ssem.at[rb],
                         add=True)

    def scatter_wait(t):
        ib = lax.rem(t, NIB)
        rb = lax.rem(t, NRB)
        pltpu.make_async_copy(rows.at[rb], agg_sh.at[dstc.at[ib]],
                              ssem.at[rb]).wait()

    for t in range(LI):
        idx_load(t)
    for k in range(WBC):
        pltpu.make_async_copy(rows.at[0],
                              agg_sh.at[pl.ds(rbase, CHUNK)],
                              gsem.at[0]).wait()
    for t in range(LG):
        idx_wait(t)
        gather_start(t)
    plsc.subcore_barrier()

    def step(t, carry):
        @pl.when(t >= LG)
        def _():
            scatter_wait(t - LG)

        @pl.when(t + LI < NCHUNK)
        def _():
            idx_load(t + LI)

        @pl.when(t + LG < NCHUNK)
        def _():
            idx_wait(t + LG)
            gather_start(t + LG)

        gather_wait(t)
        scatter_start(t)
        return carry
    lax.fori_loop(0, NCHUNK, step, 0)

    for s in range(NCHUNK - LG, NCHUNK):
        scatter_wait(s)

    plsc.subcore_barrier()

    for k in range(WBC):
        b = k % 2
        pltpu.async_copy(agg_sh.at[pl.ds(rbase + k * CHUNK, CHUNK)],
                         rows.at[b], gsem.at[b])
        pltpu.make_async_copy(agg_sh.at[pl.ds(rbase, CHUNK)],
                              rows.at[b], gsem.at[b]).wait()
        pltpu.async_copy(rows.at[b],
                         out_hbm.at[c, pl.ds(rbase + k * CHUNK, CHUNK)],
                         ssem.at[b])
        if k >= 1:
            pb = (k - 1) % 2
            pltpu.make_async_copy(rows.at[pb],
                                  out_hbm.at[c, pl.ds(rbase, CHUNK)],
                                  ssem.at[pb]).wait()
    pltpu.make_async_copy(rows.at[(WBC - 1) % 2],
                          out_hbm.at[c, pl.ds(rbase, CHUNK)],
                          ssem.at[(WBC - 1) % 2]).wait()


_sc_agg = functools.partial(
    pl.kernel,
    out_type=jax.ShapeDtypeStruct((NC, N_PAD, H), jnp.float32),
    mesh=plsc.VectorSubcoreMesh(core_axis_name="c", subcore_axis_name="s"),
    scratch_types=[
        pltpu.VMEM((NIB, CHUNK), jnp.int32),
        pltpu.VMEM((NIB, CHUNK), jnp.int32),
        pltpu.VMEM((NRB, CHUNK, H), jnp.float32),
        pltpu.VMEM_SHARED((N_PAD, H), jnp.float32),
        pltpu.SemaphoreType.DMA((NIB,)),
        pltpu.SemaphoreType.DMA((NRB,)),
        pltpu.SemaphoreType.DMA((NRB,)),
    ],
)(_sc_agg_body)


ROWS_BLK = 1000
GRID = N_NODES // ROWS_BLK
MLP_BLK = 5000
MLP_GRID = N_NODES // MLP_BLK
FW = 32


def _inproj_body(f_ref, w_ref, o_ref):
    o_ref[...] = jnp.dot(f_ref[...], w_ref[...],
                         preferred_element_type=jnp.float32)


def _mlp_body(x_ref, p_ref, w1_ref, b1_ref, w2_ref, b2_ref, o_ref):
    h = x_ref[...] + p_ref[0] + p_ref[1]
    h = jnp.maximum(jnp.dot(h, w1_ref[...],
                            preferred_element_type=jnp.float32) + b1_ref[...], 0.0)
    h = jnp.dot(h, w2_ref[...], preferred_element_type=jnp.float32) + b2_ref[...]
    o_ref[...] = jnp.maximum(h, 0.0)


def _mlp_out_body(x_ref, p_ref, w1_ref, b1_ref, w2_ref, b2_ref,
                  ow_ref, ob_ref, oa_ref, ot_ref):
    i = pl.program_id(0)
    h = x_ref[...] + p_ref[0] + p_ref[1]
    h = jnp.maximum(jnp.dot(h, w1_ref[...],
                            preferred_element_type=jnp.float32) + b1_ref[...], 0.0)
    h = jnp.dot(h, w2_ref[...], preferred_element_type=jnp.float32) + b2_ref[...]
    h = jnp.maximum(h, 0.0)
    emb = (jnp.dot(h, ow_ref[0], preferred_element_type=jnp.float32)
           + ob_ref[0])

    @pl.when(i == 0)
    def _():
        oa_ref[...] = emb

    @pl.when(i > 0)
    def _():
        ot_ref[...] = emb


def _sel(i):
    return (i > 0).astype(jnp.int32)


_inproj = pl.pallas_call(
    _inproj_body,
    grid=(1,),
    in_specs=[
        pl.BlockSpec((N_NODES, FW), lambda i: (0, 0)),
        pl.BlockSpec((FW, H), lambda i: (0, 0)),
    ],
    out_specs=pl.BlockSpec((N_NODES, H), lambda i: (0, 0)),
    out_shape=jax.ShapeDtypeStruct((N_NODES, H), jnp.float32),
)

_mlp = pl.pallas_call(
    _mlp_body,
    grid=(MLP_GRID,),
    in_specs=[
        pl.BlockSpec((MLP_BLK, H), lambda i: (i, 0)),
        pl.BlockSpec((NC, MLP_BLK, H), lambda i: (0, i, 0)),
        pl.BlockSpec((H, H), lambda i: (0, 0)),
        pl.BlockSpec((1, H), lambda i: (0, 0)),
        pl.BlockSpec((H, H), lambda i: (0, 0)),
        pl.BlockSpec((1, H), lambda i: (0, 0)),
    ],
    out_specs=pl.BlockSpec((MLP_BLK, H), lambda i: (i, 0)),
    out_shape=jax.ShapeDtypeStruct((N_NODES, H), jnp.float32),
)

_mlp_out = pl.pallas_call(
    _mlp_out_body,
    grid=(GRID,),
    in_specs=[
        pl.BlockSpec((ROWS_BLK, H), lambda i: (i, 0)),
        pl.BlockSpec((NC, ROWS_BLK, H), lambda i: (0, i, 0)),
        pl.BlockSpec((H, H), lambda i: (0, 0)),
        pl.BlockSpec((1, H), lambda i: (0, 0)),
        pl.BlockSpec((H, H), lambda i: (0, 0)),
        pl.BlockSpec((1, H), lambda i: (0, 0)),
        pl.BlockSpec((1, H, H), lambda i: (_sel(i), 0, 0)),
        pl.BlockSpec((1, 1, H), lambda i: (_sel(i), 0, 0)),
    ],
    out_specs=[
        pl.BlockSpec((N_AGENTS, H), lambda i: (0, 0)),
        pl.BlockSpec((ROWS_BLK, H), lambda i: (jnp.maximum(i - 1, 0), 0)),
    ],
    out_shape=[
        jax.ShapeDtypeStruct((N_AGENTS, H), jnp.float32),
        jax.ShapeDtypeStruct((N_TASKS, H), jnp.float32),
    ],
)


def kernel(agent_features, task_features, edge_index,
           agent_in_w, agent_in_b, task_in_w, task_in_b,
           gin_w1, gin_b1, gin_w2, gin_b2,
           agent_out_w, agent_out_b, task_out_w, task_out_b):
    src = edge_index[0]
    dst = edge_index[1]

    AF = agent_features.shape[1]
    TF = task_features.shape[1]
    fa = jnp.zeros((N_AGENTS, FW), jnp.float32)
    fa = fa.at[:, :AF].set(agent_features).at[:, AF + TF].set(1.0)
    ft = jnp.zeros((N_TASKS, FW), jnp.float32)
    ft = ft.at[:, AF:AF + TF].set(task_features).at[:, AF + TF + 1].set(1.0)
    feat = jnp.concatenate([fa, ft], axis=0)
    win = jnp.zeros((FW, H), jnp.float32)
    win = (win.at[:AF].set(agent_in_w)
              .at[AF:AF + TF].set(task_in_w)
              .at[AF + TF].set(agent_in_b)
              .at[AF + TF + 1].set(task_in_b))
    wout = jnp.stack([agent_out_w, task_out_w])
    bout = jnp.stack([agent_out_b, task_out_b])[:, None, :]

    x = _inproj(feat, win)
    for i in range(NUM_LAYERS):
        p = _sc_agg(x, src, dst)
        w1 = gin_w1[i]
        b1 = gin_b1[i][None, :]
        w2 = gin_w2[i]
        b2 = gin_b2[i][None, :]
        if i < NUM_LAYERS - 1:
            x = _mlp(x, p, w1, b1, w2, b2)
        else:
            emb_a, emb_t = _mlp_out(x, p, w1, b1, w2, b2, wout, bout)
    return (emb_a, emb_t)


# speedup vs baseline: 1.0586x; 1.0586x over previous
import functools

import jax
import jax.numpy as jnp
from jax import lax
from jax.experimental import pallas as pl
from jax.experimental.pallas import tpu as pltpu
from jax.experimental.pallas import tpu_sc as plsc

N_AGENTS = 1000
N_TASKS = 9000
N_NODES = 10000
N_EDGES = 320000
H = 128
NUM_LAYERS = 3

NC = 2
NS = 16
EDGES_PER_TILE = N_EDGES // (NC * NS)
CHUNK = 80
NCHUNK = EDGES_PER_TILE // CHUNK
N_PAD = 10240
ROWS_PER_TILE = N_PAD // NS
WB = 128
NWB = ROWS_PER_TILE // WB
NRB = 4
NIB = 8
LG = 3
LI = 6
WBC = ROWS_PER_TILE // CHUNK


def _sc_agg_body(x_hbm, src_hbm, dst_hbm, out_hbm,
                 srcc, dstc, rows, agg_sh, isem, gsem, ssem):
    c = lax.axis_index("c")
    sid = lax.axis_index("s")
    ebase = (c * NS + sid) * EDGES_PER_TILE
    rbase = sid * ROWS_PER_TILE

    z16 = jnp.zeros((16,), jnp.float32)

    def zrow(r, carry):
        for j in range(8):
            rows[0, r, pl.ds(j * 16, 16)] = z16
        return carry
    lax.fori_loop(0, CHUNK, zrow, 0)

    for k in range(WBC):
        pltpu.async_copy(rows.at[0], agg_sh.at[pl.ds(rbase + k * CHUNK, CHUNK)],
                         gsem.at[0])

    def idx_load(t):
        ib = lax.rem(t, NIB)
        pltpu.async_copy(src_hbm.at[pl.ds(ebase + t * CHUNK, CHUNK)],
                         srcc.at[ib], isem.at[ib])
        pltpu.async_copy(dst_hbm.at[pl.ds(ebase + t * CHUNK, CHUNK)],
                         dstc.at[ib], isem.at[ib])

    def idx_wait(t):
        ib = lax.rem(t, NIB)
        pltpu.make_async_copy(src_hbm.at[pl.ds(0, CHUNK)],
                              srcc.at[ib], isem.at[ib]).wait()
        pltpu.make_async_copy(dst_hbm.at[pl.ds(0, CHUNK)],
                              dstc.at[ib], isem.at[ib]).wait()

    def gather_start(t):
        ib = lax.rem(t, NIB)
        rb = lax.rem(t, NRB)
        pltpu.async_copy(x_hbm.at[srcc.at[ib]], rows.at[rb], gsem.at[rb])

    def gather_wait(t):
        rb = lax.rem(t, NRB)
        pltpu.make_async_copy(x_hbm.at[srcc.at[0]], rows.at[rb],
                              gsem.at[rb]).wait()

    def scatter_start(t):
        ib = lax.rem(t, NIB)
        rb = lax.rem(t, NRB)
        pltpu.async_copy(rows.at[rb], agg_sh.at[dstc.at[ib]], ssem.at[rb],
                         add=True)

    def scatter_wait(t):
        ib = lax.rem(t, NIB)
        rb = lax.rem(t, NRB)
        pltpu.make_async_copy(rows.at[rb], agg_sh.at[dstc.at[ib]],
                              ssem.at[rb]).wait()

    for t in range(LI):
        idx_load(t)
    for k in range(WBC):
        pltpu.make_async_copy(rows.at[0],
                              agg_sh.at[pl.ds(rbase, CHUNK)],
                              gsem.at[0]).wait()
    for t in range(LG):
        idx_wait(t)
        gather_start(t)
    plsc.subcore_barrier()

    def step(t, carry):
        @pl.when(t >= LG)
        def _():
            scatter_wait(t - LG)

        @pl.when(t + LI < NCHUNK)
        def _():
            idx_load(t + LI)

        @pl.when(t + LG < NCHUNK)
        def _():
            idx_wait(t + LG)
            gather_start(t + LG)

        gather_wait(t)
        scatter_start(t)
        return carry
    lax.fori_loop(0, NCHUNK, step, 0)

    for s in range(NCHUNK - LG, NCHUNK):
        scatter_wait(s)

    plsc.subcore_barrier()

    for k in range(WBC):
        b = k % 2
        pltpu.async_copy(agg_sh.at[pl.ds(rbase + k * CHUNK, CHUNK)],
                         rows.at[b], gsem.at[b])
        pltpu.make_async_copy(agg_sh.at[pl.ds(rbase, CHUNK)],
                              rows.at[b], gsem.at[b]).wait()
        pltpu.async_copy(rows.at[b],
                         out_hbm.at[c, pl.ds(rbase + k * CHUNK, CHUNK)],
                         ssem.at[b])
        if k >= 1:
            pb = (k - 1) % 2
            pltpu.make_async_copy(rows.at[pb],
                                  out_hbm.at[c, pl.ds(rbase, CHUNK)],
                                  ssem.at[pb]).wait()
    pltpu.make_async_copy(rows.at[(WBC - 1) % 2],
                          out_hbm.at[c, pl.ds(rbase, CHUNK)],
                          ssem.at[(WBC - 1) % 2]).wait()


_sc_agg = functools.partial(
    pl.kernel,
    out_type=jax.ShapeDtypeStruct((NC, N_PAD, H), jnp.float32),
    mesh=plsc.VectorSubcoreMesh(core_axis_name="c", subcore_axis_name="s"),
    scratch_types=[
        pltpu.VMEM((NIB, CHUNK), jnp.int32),
        pltpu.VMEM((NIB, CHUNK), jnp.int32),
        pltpu.VMEM((NRB, CHUNK, H), jnp.float32),
        pltpu.VMEM_SHARED((N_PAD, H), jnp.float32),
        pltpu.SemaphoreType.DMA((NIB,)),
        pltpu.SemaphoreType.DMA((NRB,)),
        pltpu.SemaphoreType.DMA((NRB,)),
    ],
)(_sc_agg_body)


ROWS_BLK = 1000
GRID = N_NODES // ROWS_BLK
MLP_BLK = 5000
MLP_GRID = N_NODES // MLP_BLK
FW = 32


def _inproj_body(f_ref, w_ref, o_ref):
    o_ref[...] = jnp.dot(f_ref[...], w_ref[...],
                         preferred_element_type=jnp.float32)


def _mlp_body(x_ref, p_ref, w1_ref, b1_ref, w2_ref, b2_ref, o_ref):
    h = x_ref[...] + p_ref[0] + p_ref[1]
    h = jnp.maximum(jnp.dot(h, w1_ref[...],
                            preferred_element_type=jnp.float32) + b1_ref[...], 0.0)
    h = jnp.dot(h, w2_ref[...], preferred_element_type=jnp.float32) + b2_ref[...]
    o_ref[...] = jnp.maximum(h, 0.0)


def _mlp_out_body(x_ref, p_ref, w1_ref, b1_ref, w2_ref, b2_ref,
                  ow_ref, ob_ref, oa_ref, ot_ref):
    i = pl.program_id(0)
    h = x_ref[...] + p_ref[0] + p_ref[1]
    h = jnp.maximum(jnp.dot(h, w1_ref[...],
                            preferred_element_type=jnp.float32) + b1_ref[...], 0.0)
    h = jnp.dot(h, w2_ref[...], preferred_element_type=jnp.float32) + b2_ref[...]
    h = jnp.maximum(h, 0.0)
    emb = (jnp.dot(h, ow_ref[0], preferred_element_type=jnp.float32)
           + ob_ref[0])

    @pl.when(i == 0)
    def _():
        oa_ref[...] = emb

    @pl.when(i > 0)
    def _():
        ot_ref[...] = emb


def _sel(i):
    return (i > 0).astype(jnp.int32)


_inproj = pl.pallas_call(
    _inproj_body,
    grid=(1,),
    in_specs=[
        pl.BlockSpec((N_NODES, FW), lambda i: (0, 0)),
        pl.BlockSpec((FW, H), lambda i: (0, 0)),
    ],
    out_specs=pl.BlockSpec((N_NODES, H), lambda i: (0, 0)),
    out_shape=jax.ShapeDtypeStruct((N_NODES, H), jnp.float32),
)

_mlp = pl.pallas_call(
    _mlp_body,
    grid=(MLP_GRID,),
    in_specs=[
        pl.BlockSpec((MLP_BLK, H), lambda i: (i, 0)),
        pl.BlockSpec((NC, MLP_BLK, H), lambda i: (0, i, 0)),
        pl.BlockSpec((H, H), lambda i: (0, 0)),
        pl.BlockSpec((1, H), lambda i: (0, 0)),
        pl.BlockSpec((H, H), lambda i: (0, 0)),
        pl.BlockSpec((1, H), lambda i: (0, 0)),
    ],
    out_specs=pl.BlockSpec((MLP_BLK, H), lambda i: (i, 0)),
    out_shape=jax.ShapeDtypeStruct((N_NODES, H), jnp.float32),
)

_mlp_out = pl.pallas_call(
    _mlp_out_body,
    grid=(GRID,),
    in_specs=[
        pl.BlockSpec((ROWS_BLK, H), lambda i: (i, 0)),
        pl.BlockSpec((NC, ROWS_BLK, H), lambda i: (0, i, 0)),
        pl.BlockSpec((H, H), lambda i: (0, 0)),
        pl.BlockSpec((1, H), lambda i: (0, 0)),
        pl.BlockSpec((H, H), lambda i: (0, 0)),
        pl.BlockSpec((1, H), lambda i: (0, 0)),
        pl.BlockSpec((1, H, H), lambda i: (_sel(i), 0, 0)),
        pl.BlockSpec((1, 1, H), lambda i: (_sel(i), 0, 0)),
    ],
    out_specs=[
        pl.BlockSpec((N_AGENTS, H), lambda i: (0, 0)),
        pl.BlockSpec((ROWS_BLK, H), lambda i: (jnp.maximum(i - 1, 0), 0)),
    ],
    out_shape=[
        jax.ShapeDtypeStruct((N_AGENTS, H), jnp.float32),
        jax.ShapeDtypeStruct((N_TASKS, H), jnp.float32),
    ],
)


def kernel(agent_features, task_features, edge_index,
           agent_in_w, agent_in_b, task_in_w, task_in_b,
           gin_w1, gin_b1, gin_w2, gin_b2,
           agent_out_w, agent_out_b, task_out_w, task_out_b):
    src = edge_index[0]
    dst = edge_index[1]

    AF = agent_features.shape[1]
    TF = task_features.shape[1]
    fa = jnp.zeros((N_AGENTS, FW), jnp.float32)
    fa = fa.at[:, :AF].set(agent_features).at[:, AF + TF].set(1.0)
    ft = jnp.zeros((N_TASKS, FW), jnp.float32)
    ft = ft.at[:, AF:AF + TF].set(task_features).at[:, AF + TF + 1].set(1.0)
    feat = jnp.concatenate([fa, ft], axis=0)
    win = jnp.zeros((FW, H), jnp.float32)
    win = (win.at[:AF].set(agent_in_w)
              .at[AF:AF + TF].set(task_in_w)
              .at[AF + TF].set(agent_in_b)
              .at[AF + TF + 1].set(task_in_b))
    wout = jnp.stack([agent_out_w, task_out_w])
    bout = jnp.stack([agent_out_b, task_out_b])[:, None, :]

    x = _inproj(feat, win)
    for i in range(NUM_LAYERS):
        p = _sc_agg(x, src, dst)
        w1 = gin_w1[i]
        b1 = gin_b1[i][None, :]
        w2 = gin_w2[i]
        b2 = gin_b2[i][None, :]
        if i < NUM_LAYERS - 1:
            x = _mlp(x, p, w1, b1, w2, b2)
        else:
            emb_a, emb_t = _mlp_out(x, p, w1, b1, w2, b2, wout, bout)
    return (emb_a, emb_t)
